# baseline (speedup 1.0000x reference)
import functools

import jax
import jax.numpy as jnp
from jax import lax
from jax.experimental import pallas as pl
from jax.experimental.pallas import tpu as pltpu
from jax.experimental.pallas import tpu_sc as plsc

_N = 4096
_K = 30
_CHUNK = 32
_NCHUNK = _N // _CHUNK
_TOPC = 32
_NEG = -1e15
_R = 256
_NW = 32
_ROWS_PER_W = _N // _NW
_CHUNKS_PER_W = _ROWS_PER_W * _TOPC


def _stage1_kernel(x_ref, idx_ref):
    x = x_ref[...]
    cm = jnp.max(x.reshape(_R, _NCHUNK, _CHUNK), axis=2)
    lane = lax.broadcasted_iota(jnp.int32, (_R, _NCHUNK), 1)
    i = pl.program_id(0)
    rowbase = (i * _R + lax.broadcasted_iota(jnp.int32, (_R, 1), 0)) * _NCHUNK
    work = cm
    cols = []
    for _ in range(_TOPC):
        m = jnp.max(work, axis=1, keepdims=True)
        hit = work == m
        cid = jnp.min(jnp.where(hit, lane, jnp.int32(_NCHUNK)), axis=1,
                      keepdims=True)
        cols.append(rowbase + cid)
        work = jnp.where(lane == cid, -jnp.inf, work)
    idx_ref[...] = jnp.concatenate(cols, axis=1)


def _stage3_kernel(x_ref, g_ref, o_ref):
    g = g_ref[...]
    gb = lax.bitcast_convert_type(g, jnp.int32)
    keyg = gb ^ jnp.bitwise_and(jnp.right_shift(gb, 31), jnp.int32(0x7FFFFFFF))
    min32 = jnp.int32(-(2**31))

    def body(i, w):
        bit = jnp.left_shift(jnp.int32(1), jnp.int32(31) - i)
        cand_w = jnp.bitwise_or(w, bit)
        cand_t = jnp.bitwise_xor(cand_w, min32)
        cnt = jnp.sum((keyg >= cand_t).astype(jnp.float32), axis=1,
                      keepdims=True)
        return jnp.where(cnt >= _K, cand_w, w)

    w0 = jnp.zeros((_R, 1), jnp.int32)
    w = lax.fori_loop(0, 32, body, w0)
    kth = jnp.bitwise_xor(w, min32)

    x = x_ref[...]
    bi = lax.bitcast_convert_type(x, jnp.int32)
    key = bi ^ jnp.bitwise_and(jnp.right_shift(bi, 31), jnp.int32(0x7FFFFFFF))
    keep = (key >= kth) & (x > 0.0)
    m = jnp.where(keep, x, _NEG)
    rowmax = jnp.max(m, axis=1, keepdims=True)
    e = jnp.exp(m - rowmax)
    s = jnp.sum(e, axis=1, keepdims=True)
    o_ref[...] = e / s


@functools.lru_cache(maxsize=1)
def _make_sc_gather():
    @functools.partial(
        pl.kernel,
        mesh=plsc.VectorSubcoreMesh(core_axis_name="c", subcore_axis_name="s"),
        compiler_params=pltpu.CompilerParams(use_tc_tiling_on_sc=False),
        out_type=jax.ShapeDtypeStruct((_N * _TOPC, _CHUNK), jnp.float32),
        scratch_types=[
            pltpu.VMEM((_TOPC, 128), jnp.int32),
            pltpu.VMEM((_CHUNKS_PER_W // 2, _CHUNK), jnp.float32),
            pltpu.SemaphoreType.DMA,
        ],
    )
    def _sc_gather(table_hbm, idx_hbm, out_hbm, idx_v, rows_v, sem):
        wid = lax.axis_index("s") * 2 + lax.axis_index("c")
        pltpu.sync_copy(idx_hbm.at[wid], idx_v)
        half_chunks = _CHUNKS_PER_W // 2
        for half in range(2):
            copies = []
            for j in range(16):
                jj = half * 16 + j
                copies.append(
                    pltpu.async_copy(
                        table_hbm.at[idx_v.at[jj]],
                        rows_v.at[pl.ds(j * 128, 128)],
                        sem,
                    )
                )
            for c in copies:
                c.wait()
            pltpu.sync_copy(
                rows_v,
                out_hbm.at[pl.ds(wid * _CHUNKS_PER_W + half * half_chunks,
                                 half_chunks)],
            )

    return _sc_gather


def kernel(new_supports):
    x = new_supports
    idx = pl.pallas_call(
        _stage1_kernel,
        grid=(_N // _R,),
        in_specs=[pl.BlockSpec((_R, _N), lambda i: (i, 0))],
        out_specs=pl.BlockSpec((_R, _TOPC), lambda i: (i, 0)),
        out_shape=jax.ShapeDtypeStruct((_N, _TOPC), jnp.int32),
    )(x)
    table = x.reshape(_N * _NCHUNK, _CHUNK)
    idx_sc = idx.reshape(_NW, _TOPC, 128)
    g = _make_sc_gather()(table, idx_sc)
    g = g.reshape(_N, _TOPC * _CHUNK)
    return pl.pallas_call(
        _stage3_kernel,
        grid=(_N // _R,),
        in_specs=[
            pl.BlockSpec((_R, _N), lambda i: (i, 0)),
            pl.BlockSpec((_R, _TOPC * _CHUNK), lambda i: (i, 0)),
        ],
        out_specs=pl.BlockSpec((_R, _N), lambda i: (i, 0)),
        out_shape=jax.ShapeDtypeStruct((_N, _N), jnp.float32),
    )(x, g)
